# Initial kernel scaffold; baseline (speedup 1.0000x reference)
import jax
import jax.numpy as jnp
from jax.experimental import pallas as pl


def kernel(x, edge_index, label, eps_pos, eps_neg, weight_self, weight_pos, weight_neg):
    raise NotImplementedError("write your pallas kernel here")


# speedup vs baseline: 9.4017x; 9.4017x over previous
import functools

import jax
import jax.numpy as jnp
from jax import lax
from jax.experimental import pallas as pl
from jax.experimental.pallas import tpu as pltpu
from jax.experimental.pallas import tpu_sc as plsc

N_NODES = 10000
N_EDGES = 320000
D = 128
DH = 64
NSUB = 16
NCORE = 2
CHUNK = 128
EPT = 20096
NCHUNK = EPT // CHUNK
E_PAD = NSUB * EPT
ACC_ROWS = 10240
ROWS_PER_TILE = ACC_ROWS // NSUB


def _sc_segment_sums(xa, xb, src, dst, label_i32, zeros):
    mesh = plsc.VectorSubcoreMesh(core_axis_name="c", subcore_axis_name="s")

    @functools.partial(
        pl.kernel,
        out_type=[
            jax.ShapeDtypeStruct((NCORE, ACC_ROWS, DH), jnp.float32),
            jax.ShapeDtypeStruct((NCORE, ACC_ROWS, DH), jnp.float32),
        ],
        mesh=mesh,
        compiler_params=pltpu.CompilerParams(use_tc_tiling_on_sc=False),
        scratch_types=[
            pltpu.VMEM((CHUNK,), jnp.int32),
            pltpu.VMEM((CHUNK,), jnp.int32),
            pltpu.VMEM((CHUNK,), jnp.int32),
            pltpu.VMEM((CHUNK,), jnp.int32),
            pltpu.VMEM((CHUNK, DH), jnp.float32),
            pltpu.VMEM_SHARED((ACC_ROWS, DH), jnp.float32),
            pltpu.VMEM_SHARED((ACC_ROWS, DH), jnp.float32),
            pltpu.SemaphoreType.DMA,
            pltpu.SemaphoreType.DMA,
        ],
    )
    def seg(xa_hbm, xb_hbm, src_hbm, dst_hbm, lbl_hbm, z_hbm,
            s0_hbm, s1_hbm,
            src_v, dst_v, dst1_v, lblc_v, rows_v, acc0, acc1, sem, sem2):
        c = lax.axis_index("c")
        s = lax.axis_index("s")
        trash = N_NODES + s

        def run(x_hbm):
            z0 = s * ROWS_PER_TILE
            pltpu.sync_copy(z_hbm.at[pl.ds(z0, ROWS_PER_TILE)],
                            acc0.at[pl.ds(z0, ROWS_PER_TILE)])
            pltpu.sync_copy(z_hbm.at[pl.ds(z0, ROWS_PER_TILE)],
                            acc1.at[pl.ds(z0, ROWS_PER_TILE)])
            plsc.subcore_barrier()

            base = s * EPT

            def chunk(i, carry):
                off = base + i * CHUNK
                pltpu.sync_copy(src_hbm.at[pl.ds(off, CHUNK)], src_v)
                pltpu.sync_copy(dst_hbm.at[pl.ds(off, CHUNK)], dst_v)
                rows_cp = pltpu.async_copy(x_hbm.at[src_v], rows_v, sem)
                lbl_cp = pltpu.async_copy(lbl_hbm.at[src_v], lblc_v, sem2)
                lbl_cp.wait()
                for j in range(CHUNK // 16):
                    sl = pl.ds(j * 16, 16)
                    dst1_v[sl] = jnp.where(lblc_v[sl] == 1, dst_v[sl], trash)
                rows_cp.wait()
                pltpu.sync_copy(rows_v, acc0.at[dst_v], add=True)
                pltpu.sync_copy(rows_v, acc1.at[dst1_v], add=True)
                return carry

            lax.fori_loop(0, NCHUNK, chunk, 0)
            plsc.subcore_barrier()

            r0 = s * ROWS_PER_TILE
            pltpu.sync_copy(acc0.at[pl.ds(r0, ROWS_PER_TILE)],
                            s0_hbm.at[c, pl.ds(r0, ROWS_PER_TILE)])
            pltpu.sync_copy(acc1.at[pl.ds(r0, ROWS_PER_TILE)],
                            s1_hbm.at[c, pl.ds(r0, ROWS_PER_TILE)])

        @pl.when(c == 0)
        def _():
            run(xa_hbm)

        @pl.when(c == 1)
        def _():
            run(xb_hbm)

    return seg(xa, xb, src, dst, label_i32, zeros)


def _tc_epilogue(eps2, x, s0, s1, ws, wp, wn):
    BN = 2000
    grid = N_NODES // BN

    def body(eps_ref, x_ref, s0_ref, s1_ref, ws_ref, wp_ref, wn_ref, o_ref):
        ep = eps_ref[0]
        en = eps_ref[1]
        s0a = s0_ref[0]
        s0b = s0_ref[1]
        s1a = s1_ref[0]
        s1b = s1_ref[1]
        pa = s0a + s1a
        pb = s0b + s1b
        xb = x_ref[...]
        hfa = xb[:, :DH] + (1.0 + ep) * pa + (1.0 + en) * s1a
        hfb = xb[:, DH:] + (1.0 + ep) * pb + (1.0 + en) * s1b

        def mm(a, b, w_ref):
            return (jnp.dot(a, w_ref[:DH, :], preferred_element_type=jnp.float32)
                    + jnp.dot(b, w_ref[DH:, :], preferred_element_type=jnp.float32))

        o_ref[:, 0:D] = jnp.maximum(mm(hfa, hfb, ws_ref), 0.0)
        o_ref[:, D:2 * D] = jnp.maximum(mm(pa, pb, wp_ref), 0.0)
        o_ref[:, 2 * D:3 * D] = jnp.maximum(mm(s1a, s1b, wn_ref), 0.0)

    return pl.pallas_call(
        body,
        grid=(grid,),
        in_specs=[
            pl.BlockSpec(memory_space=pltpu.SMEM),
            pl.BlockSpec((BN, D), lambda i: (i, 0)),
            pl.BlockSpec((NCORE, BN, DH), lambda i: (0, i, 0)),
            pl.BlockSpec((NCORE, BN, DH), lambda i: (0, i, 0)),
            pl.BlockSpec((D, D), lambda i: (0, 0)),
            pl.BlockSpec((D, D), lambda i: (0, 0)),
            pl.BlockSpec((D, D), lambda i: (0, 0)),
        ],
        out_specs=pl.BlockSpec((BN, 3 * D), lambda i: (i, 0)),
        out_shape=jax.ShapeDtypeStruct((N_NODES, 3 * D), jnp.float32),
    )(eps2, x, s0, s1, ws, wp, wn)


def kernel(x, edge_index, label, eps_pos, eps_neg, weight_self, weight_pos, weight_neg):
    src = edge_index[0].astype(jnp.int32)
    dst = edge_index[1].astype(jnp.int32)
    lbl = label.astype(jnp.int32)

    pad = E_PAD - N_EDGES
    src_p = jnp.concatenate([src, jnp.zeros((pad,), jnp.int32)])
    dst_p = jnp.concatenate([dst, jnp.full((pad,), N_NODES, jnp.int32)])

    xa = x[:, :DH]
    xb = x[:, DH:]
    zeros = jnp.zeros((ACC_ROWS, DH), jnp.float32)

    s0, s1 = _sc_segment_sums(xa, xb, src_p, dst_p, lbl, zeros)

    eps2 = jnp.concatenate([eps_pos, eps_neg])
    return _tc_epilogue(eps2, x, s0, s1, weight_self, weight_pos, weight_neg)
